# speedup vs baseline: 2.1511x; 2.1511x over previous
import functools

import jax
import jax.numpy as jnp
from jax import lax
from jax.experimental import pallas as pl
from jax.experimental.pallas import tpu as pltpu

T, H, I, E, A, R, K = 2048, 768, 1536, 8, 4, 16, 2
BLK = 128
NB = (T * K) // BLK + E - 1
NB = NB + (-NB) % 2
NPAD = NB * BLK


def _tc_body(be_ref, x_ref, pw_ref, wgu_ref, wd_ref, ga_ref, gb_ref,
             ua_ref, ub_ref, da_ref, db_ref, out_ref):
    del be_ref
    xb = x_ref[...]
    gu = jnp.dot(xb, wgu_ref[0], preferred_element_type=jnp.float32)
    lg = jnp.dot(jnp.dot(xb, ga_ref[0], preferred_element_type=jnp.float32),
                 gb_ref[0], preferred_element_type=jnp.float32)
    lu = jnp.dot(jnp.dot(xb, ua_ref[0], preferred_element_type=jnp.float32),
                 ub_ref[0], preferred_element_type=jnp.float32)
    gate = gu[:, :I] + lg
    up = gu[:, I:] + lu
    act = gate * jax.nn.sigmoid(gate) * up
    dn = jnp.dot(act, wd_ref[0], preferred_element_type=jnp.float32)
    ld = jnp.dot(jnp.dot(act, da_ref[0], preferred_element_type=jnp.float32),
                 db_ref[0], preferred_element_type=jnp.float32)
    out_ref[...] = (dn + ld) * pw_ref[...]


def kernel(hidden_states, topk_ids, topk_weights, gate_a, gate_b, up_a, up_b,
           down_a, down_b, weight_indices, seq_lens, lora_ranks, scalings,
           base_gate_up_weight, base_down_weight):
    del seq_lens, lora_ranks
    x = hidden_states.astype(jnp.float32)
    adapter = weight_indices[0]
    scaling = scalings[adapter].astype(jnp.float32)
    ga = jnp.transpose(gate_a[adapter], (0, 2, 1)).astype(jnp.float32)
    gb = jnp.transpose(gate_b[adapter], (0, 2, 1)).astype(jnp.float32) * scaling
    ua = jnp.transpose(up_a[adapter], (0, 2, 1)).astype(jnp.float32)
    ub = jnp.transpose(up_b[adapter], (0, 2, 1)).astype(jnp.float32) * scaling
    da = jnp.transpose(down_a[adapter], (0, 2, 1)).astype(jnp.float32)
    db = jnp.transpose(down_b[adapter], (0, 2, 1)).astype(jnp.float32) * scaling

    flat_ids = topk_ids.reshape(-1)
    flat_w = topk_weights.reshape(-1).astype(jnp.float32)
    oh = (flat_ids[:, None] == jnp.arange(E, dtype=jnp.int32)[None, :])
    g = jnp.sum(oh, axis=0, dtype=jnp.int32)
    blocks_e = (g + BLK - 1) // BLK
    pad_start = jnp.concatenate([jnp.zeros((1,), jnp.int32),
                                 jnp.cumsum(blocks_e)[:-1].astype(jnp.int32)]) * BLK
    rank = jnp.take_along_axis(jnp.cumsum(oh, axis=0, dtype=jnp.int32) - 1,
                               flat_ids[:, None].astype(jnp.int32), 1)[:, 0]
    dest = pad_start[flat_ids] + rank
    tok_of_slot = (jnp.arange(T * K, dtype=jnp.int32) // K)
    ptok = jnp.zeros((NPAD,), jnp.int32).at[dest].set(tok_of_slot)
    pw = jnp.zeros((NPAD,), jnp.float32).at[dest].set(flat_w)
    block_expert = jnp.minimum(
        jnp.searchsorted(jnp.cumsum(blocks_e), jnp.arange(NB, dtype=jnp.int32),
                         side="right"),
        E - 1).astype(jnp.int32)
    pos = dest.reshape(T, K)

    sorted_x = x[ptok]

    grid_spec = pltpu.PrefetchScalarGridSpec(
        num_scalar_prefetch=1,
        grid=(NB,),
        in_specs=[
            pl.BlockSpec((BLK, H), lambda b, be: (b, 0)),
            pl.BlockSpec((BLK, 1), lambda b, be: (b, 0)),
            pl.BlockSpec((1, H, 2 * I), lambda b, be: (be[b], 0, 0)),
            pl.BlockSpec((1, I, H), lambda b, be: (be[b], 0, 0)),
            pl.BlockSpec((1, H, R), lambda b, be: (be[b], 0, 0)),
            pl.BlockSpec((1, R, I), lambda b, be: (be[b], 0, 0)),
            pl.BlockSpec((1, H, R), lambda b, be: (be[b], 0, 0)),
            pl.BlockSpec((1, R, I), lambda b, be: (be[b], 0, 0)),
            pl.BlockSpec((1, I, R), lambda b, be: (be[b], 0, 0)),
            pl.BlockSpec((1, R, H), lambda b, be: (be[b], 0, 0)),
        ],
        out_specs=pl.BlockSpec((BLK, H), lambda b, be: (b, 0)),
    )
    sorted_out = pl.pallas_call(
        _tc_body,
        grid_spec=grid_spec,
        out_shape=jax.ShapeDtypeStruct((NPAD, H), jnp.float32),
        compiler_params=pltpu.CompilerParams(
            vmem_limit_bytes=100 * 1024 * 1024),
    )(block_expert, sorted_x, pw.reshape(NPAD, 1), base_gate_up_weight,
      base_down_weight, ga, gb, ua, ub, da, db)

    out = sorted_out[pos[:, 0]] + sorted_out[pos[:, 1]]
    return out.astype(hidden_states.dtype)


# speedup vs baseline: 2.3243x; 1.0805x over previous
import functools

import jax
import jax.numpy as jnp
from jax import lax
from jax.experimental import pallas as pl
from jax.experimental.pallas import tpu as pltpu
from jax.experimental.pallas import tpu_sc as plsc

T, H, I, E, A, R, K = 2048, 768, 1536, 8, 4, 16, 2
BLK = 128
NB = (T * K) // BLK + E - 1
NB = NB + (-NB) % 2
NPAD = NB * BLK

NW = 32
G_PER_W = NPAD // NW
G_CHUNK = 80
C_PER_W = T // NW


def _wid():
    return lax.axis_index("s") * 2 + lax.axis_index("c")


def _sc_gather_body(x_hbm, idx_hbm, out_hbm, idx_v, rows_v, sem):
    base = _wid() * G_PER_W
    for c in range(G_PER_W // G_CHUNK):
        off = base + c * G_CHUNK
        pltpu.sync_copy(idx_hbm.at[pl.ds(off, G_CHUNK)], idx_v)
        pltpu.async_copy(x_hbm.at[idx_v], rows_v, sem).wait()
        pltpu.sync_copy(rows_v, out_hbm.at[pl.ds(off, G_CHUNK)])


def _sc_combine_body(so_hbm, p0_hbm, p1_hbm, out_hbm, i0_v, i1_v, a_v, b_v, sem):
    base = _wid() * C_PER_W
    pltpu.sync_copy(p0_hbm.at[pl.ds(base, C_PER_W)], i0_v)
    pltpu.sync_copy(p1_hbm.at[pl.ds(base, C_PER_W)], i1_v)
    pltpu.async_copy(so_hbm.at[i0_v], a_v, sem).wait()
    pltpu.async_copy(so_hbm.at[i1_v], b_v, sem).wait()

    def row(r, _):
        for c in range(H // 16):
            s = pl.ds(c * 16, 16)
            a_v[r, s] = a_v[r, s] + b_v[r, s]
        return 0

    lax.fori_loop(0, C_PER_W, row, 0)
    pltpu.sync_copy(a_v, out_hbm.at[pl.ds(base, C_PER_W)])


def _sc_gather(x, ptok):
    mesh = plsc.VectorSubcoreMesh(core_axis_name="c", subcore_axis_name="s")
    f = functools.partial(
        pl.kernel, mesh=mesh,
        out_type=jax.ShapeDtypeStruct((NPAD, H), jnp.float32),
        scratch_types=[
            pltpu.VMEM((G_CHUNK,), jnp.int32),
            pltpu.VMEM((G_CHUNK, H), jnp.float32),
            pltpu.SemaphoreType.DMA,
        ])(_sc_gather_body)
    return f(x, ptok)


def _sc_combine(sorted_out, p0, p1):
    mesh = plsc.VectorSubcoreMesh(core_axis_name="c", subcore_axis_name="s")
    f = functools.partial(
        pl.kernel, mesh=mesh,
        out_type=jax.ShapeDtypeStruct((T, H), jnp.float32),
        scratch_types=[
            pltpu.VMEM((C_PER_W,), jnp.int32),
            pltpu.VMEM((C_PER_W,), jnp.int32),
            pltpu.VMEM((C_PER_W, H), jnp.float32),
            pltpu.VMEM((C_PER_W, H), jnp.float32),
            pltpu.SemaphoreType.DMA,
        ])(_sc_combine_body)
    return f(sorted_out, p0, p1)


def _tc_body(be_ref, x_ref, pw_ref, wgu_ref, wd_ref, ga_ref, gb_ref,
             ua_ref, ub_ref, da_ref, db_ref, out_ref):
    del be_ref
    xb = x_ref[...]
    gu = jnp.dot(xb, wgu_ref[0], preferred_element_type=jnp.float32)
    lg = jnp.dot(jnp.dot(xb, ga_ref[0], preferred_element_type=jnp.float32),
                 gb_ref[0], preferred_element_type=jnp.float32)
    lu = jnp.dot(jnp.dot(xb, ua_ref[0], preferred_element_type=jnp.float32),
                 ub_ref[0], preferred_element_type=jnp.float32)
    gate = gu[:, :I] + lg
    up = gu[:, I:] + lu
    act = gate * jax.nn.sigmoid(gate) * up
    dn = jnp.dot(act, wd_ref[0], preferred_element_type=jnp.float32)
    ld = jnp.dot(jnp.dot(act, da_ref[0], preferred_element_type=jnp.float32),
                 db_ref[0], preferred_element_type=jnp.float32)
    out_ref[...] = (dn + ld) * pw_ref[...]


def kernel(hidden_states, topk_ids, topk_weights, gate_a, gate_b, up_a, up_b,
           down_a, down_b, weight_indices, seq_lens, lora_ranks, scalings,
           base_gate_up_weight, base_down_weight):
    del seq_lens, lora_ranks
    x = hidden_states.astype(jnp.float32)
    adapter = weight_indices[0]
    scaling = scalings[adapter].astype(jnp.float32)
    ga = jnp.transpose(gate_a[adapter], (0, 2, 1)).astype(jnp.float32)
    gb = jnp.transpose(gate_b[adapter], (0, 2, 1)).astype(jnp.float32) * scaling
    ua = jnp.transpose(up_a[adapter], (0, 2, 1)).astype(jnp.float32)
    ub = jnp.transpose(up_b[adapter], (0, 2, 1)).astype(jnp.float32) * scaling
    da = jnp.transpose(down_a[adapter], (0, 2, 1)).astype(jnp.float32)
    db = jnp.transpose(down_b[adapter], (0, 2, 1)).astype(jnp.float32) * scaling

    flat_ids = topk_ids.reshape(-1)
    flat_w = topk_weights.reshape(-1).astype(jnp.float32)
    oh = (flat_ids[:, None] == jnp.arange(E, dtype=jnp.int32)[None, :])
    g = jnp.sum(oh, axis=0, dtype=jnp.int32)
    blocks_e = (g + BLK - 1) // BLK
    pad_start = jnp.concatenate([jnp.zeros((1,), jnp.int32),
                                 jnp.cumsum(blocks_e)[:-1].astype(jnp.int32)]) * BLK
    rank = jnp.take_along_axis(jnp.cumsum(oh, axis=0, dtype=jnp.int32) - 1,
                               flat_ids[:, None].astype(jnp.int32), 1)[:, 0]
    dest = pad_start[flat_ids] + rank
    tok_of_slot = (jnp.arange(T * K, dtype=jnp.int32) // K)
    ptok = jnp.zeros((NPAD,), jnp.int32).at[dest].set(tok_of_slot)
    pw = jnp.zeros((NPAD,), jnp.float32).at[dest].set(flat_w)
    block_expert = jnp.minimum(
        jnp.searchsorted(jnp.cumsum(blocks_e), jnp.arange(NB, dtype=jnp.int32),
                         side="right"),
        E - 1).astype(jnp.int32)
    pos = dest.reshape(T, K)

    sorted_x = _sc_gather(x, ptok)

    grid_spec = pltpu.PrefetchScalarGridSpec(
        num_scalar_prefetch=1,
        grid=(NB,),
        in_specs=[
            pl.BlockSpec((BLK, H), lambda b, be: (b, 0)),
            pl.BlockSpec((BLK, 1), lambda b, be: (b, 0)),
            pl.BlockSpec((1, H, 2 * I), lambda b, be: (be[b], 0, 0)),
            pl.BlockSpec((1, I, H), lambda b, be: (be[b], 0, 0)),
            pl.BlockSpec((1, H, R), lambda b, be: (be[b], 0, 0)),
            pl.BlockSpec((1, R, I), lambda b, be: (be[b], 0, 0)),
            pl.BlockSpec((1, H, R), lambda b, be: (be[b], 0, 0)),
            pl.BlockSpec((1, R, I), lambda b, be: (be[b], 0, 0)),
            pl.BlockSpec((1, I, R), lambda b, be: (be[b], 0, 0)),
            pl.BlockSpec((1, R, H), lambda b, be: (be[b], 0, 0)),
        ],
        out_specs=pl.BlockSpec((BLK, H), lambda b, be: (b, 0)),
    )
    sorted_out = pl.pallas_call(
        _tc_body,
        grid_spec=grid_spec,
        out_shape=jax.ShapeDtypeStruct((NPAD, H), jnp.float32),
        compiler_params=pltpu.CompilerParams(
            vmem_limit_bytes=100 * 1024 * 1024),
    )(block_expert, sorted_x, pw.reshape(NPAD, 1), base_gate_up_weight,
      base_down_weight, ga, gb, ua, ub, da, db)

    out = _sc_combine(sorted_out, pos[:, 0], pos[:, 1])
    return out.astype(hidden_states.dtype)
